# baseline (speedup 1.0000x reference)
import functools

import jax
import jax.numpy as jnp
from jax import lax
from jax.experimental import pallas as pl
from jax.experimental.pallas import tpu as pltpu
from jax.experimental.pallas import tpu_sc as plsc

B = 16384
D = 64
NUM_CORES = 2
NUM_SUBCORES = 16
NW = NUM_CORES * NUM_SUBCORES
BPW = B // NW
NCHUNK = 4
CHUNK = BPW // NCHUNK
LANES = 16


def _body(tu_hbm, ti_hbm, u_hbm, i_hbm, out_hbm,
          uidx_v, iidx_v, urows_v, irows_v, out_v, sem_u, sem_i):
    wid = lax.axis_index("s") * NUM_CORES + lax.axis_index("c")
    base = wid * BPW

    for j in range(NCHUNK):
        pltpu.sync_copy(tu_hbm.at[pl.ds(base + j * CHUNK, CHUNK)], uidx_v.at[j])
        pltpu.sync_copy(ti_hbm.at[pl.ds(base + j * CHUNK, CHUNK)], iidx_v.at[j])

    copies = []
    for j in range(NCHUNK):
        copies.append(pltpu.async_copy(
            u_hbm.at[uidx_v.at[j]], urows_v.at[pl.ds(j * CHUNK, CHUNK)], sem_u))
        copies.append(pltpu.async_copy(
            i_hbm.at[iidx_v.at[j]], irows_v.at[pl.ds(j * CHUNK, CHUNK)], sem_i))
    for c in copies:
        c.wait()

    lane = lax.iota(jnp.int32, LANES)

    def row_block(step, carry):
        rows = lane + step * LANES
        acc = jnp.zeros((LANES,), jnp.float32)
        for d in range(D):
            col = jnp.full((LANES,), d, jnp.int32)
            uv = plsc.load_gather(urows_v, [rows, col])
            iv = plsc.load_gather(irows_v, [rows, col])
            acc = acc + uv * iv
        out_v[pl.ds(step * LANES, LANES)] = acc
        return carry

    lax.fori_loop(0, BPW // LANES, row_block, 0)

    pltpu.sync_copy(out_v, out_hbm.at[pl.ds(base, BPW)])


@functools.partial(
    pl.kernel,
    out_type=jax.ShapeDtypeStruct((B,), jnp.float32),
    mesh=plsc.VectorSubcoreMesh(core_axis_name="c", subcore_axis_name="s"),
    compiler_params=pltpu.CompilerParams(
        needs_layout_passes=False, use_tc_tiling_on_sc=False),
    scratch_types=[
        pltpu.VMEM((NCHUNK, CHUNK), jnp.int32),
        pltpu.VMEM((NCHUNK, CHUNK), jnp.int32),
        pltpu.VMEM((BPW, D), jnp.float32),
        pltpu.VMEM((BPW, D), jnp.float32),
        pltpu.VMEM((BPW,), jnp.float32),
        pltpu.SemaphoreType.DMA,
        pltpu.SemaphoreType.DMA,
    ],
)
def _mf_sc(tu_hbm, ti_hbm, u_hbm, i_hbm, out_hbm,
           uidx_v, iidx_v, urows_v, irows_v, out_v, sem_u, sem_i):
    _body(tu_hbm, ti_hbm, u_hbm, i_hbm, out_hbm,
          uidx_v, iidx_v, urows_v, irows_v, out_v, sem_u, sem_i)


def kernel(Tu, Ti, uY, iY):
    return _mf_sc(Tu.astype(jnp.int32), Ti.astype(jnp.int32), uY, iY)


# speedup vs baseline: 2.2554x; 2.2554x over previous
import functools

import jax
import jax.numpy as jnp
from jax import lax
from jax.experimental import pallas as pl
from jax.experimental.pallas import tpu as pltpu
from jax.experimental.pallas import tpu_sc as plsc

B = 16384
D = 64
SLAB = 8
NSLAB = 1000000 // SLAB
NUM_CORES = 2
NUM_SUBCORES = 16
NW = NUM_CORES * NUM_SUBCORES
BPW = B // NW
IC = 4
ICHUNK = BPW // IC
LANES = 16
NR = BPW // LANES


def _body(tu_hbm, ti_hbm, u_hbm, i_hbm, out_hbm,
          tidx_v, sub_v, ubuf_v, ibuf_v, out_v, sem_u, sem_i):
    wid = lax.axis_index("s") * NUM_CORES + lax.axis_index("c")
    base = wid * BPW

    for t, idx_hbm in ((0, tu_hbm), (1, ti_hbm)):
        for j in range(IC):
            pltpu.sync_copy(idx_hbm.at[pl.ds(base + j * ICHUNK, ICHUNK)],
                            tidx_v.at[t, pl.ds(j * ICHUNK, ICHUNK)])
    for t in range(2):
        for k in range(BPW // LANES):
            flat = k * LANES
            v = tidx_v[t, pl.ds(flat, LANES)]
            sub_v[t, pl.ds(flat, LANES)] = lax.bitwise_and(v, 7)
            tidx_v[t, pl.ds(flat, LANES)] = lax.shift_right_logical(v, 3)
    lane = lax.iota(jnp.int32, LANES)

    def fire_round(r, parity):
        uslabs = tidx_v[0, pl.ds(r * LANES, LANES)]
        islabs = tidx_v[1, pl.ds(r * LANES, LANES)]
        for k in range(LANES):
            pltpu.async_copy(u_hbm.at[uslabs[k]], ubuf_v.at[parity, k], sem_u)
            pltpu.async_copy(i_hbm.at[islabs[k]], ibuf_v.at[parity, k], sem_i)

    def drain_round(parity):
        for k in range(LANES):
            pltpu.make_async_copy(u_hbm.at[0], ubuf_v.at[parity, k],
                                  sem_u).wait()
            pltpu.make_async_copy(i_hbm.at[0], ibuf_v.at[parity, k],
                                  sem_i).wait()

    def compute_round(r, parity):
        pvec = jnp.full((LANES,), parity, jnp.int32)
        usub = sub_v[0, pl.ds(r * LANES, LANES)]
        isub = sub_v[1, pl.ds(r * LANES, LANES)]
        acc = jnp.zeros((LANES,), jnp.float32)
        for d in range(D):
            col = jnp.full((LANES,), d, jnp.int32)
            uv = plsc.load_gather(ubuf_v, [pvec, lane, usub, col])
            iv = plsc.load_gather(ibuf_v, [pvec, lane, isub, col])
            acc = acc + uv * iv
        out_v[pl.ds(r * LANES, LANES)] = acc

    fire_round(0, 0)

    def loop_body(r, carry):
        parity = lax.rem(r, 2)
        fire_round(r, parity)
        drain_round(1 - parity)
        compute_round(r - 1, 1 - parity)
        return carry

    lax.fori_loop(1, NR, loop_body, 0)

    drain_round((NR - 1) % 2)
    compute_round(NR - 1, (NR - 1) % 2)

    pltpu.sync_copy(out_v, out_hbm.at[pl.ds(base, BPW)])


@functools.partial(
    pl.kernel,
    out_type=jax.ShapeDtypeStruct((B,), jnp.float32),
    mesh=plsc.VectorSubcoreMesh(core_axis_name="c", subcore_axis_name="s"),
    compiler_params=pltpu.CompilerParams(needs_layout_passes=False),
    scratch_types=[
        pltpu.VMEM((2, BPW), jnp.int32),
        pltpu.VMEM((2, BPW), jnp.int32),
        pltpu.VMEM((2, LANES, SLAB, D), jnp.float32),
        pltpu.VMEM((2, LANES, SLAB, D), jnp.float32),
        pltpu.VMEM((BPW,), jnp.float32),
        pltpu.SemaphoreType.DMA,
        pltpu.SemaphoreType.DMA,
    ],
)
def _mf_sc(tu_hbm, ti_hbm, u_hbm, i_hbm, out_hbm,
           tidx_v, sub_v, ubuf_v, ibuf_v, out_v, sem_u, sem_i):
    _body(tu_hbm, ti_hbm, u_hbm, i_hbm, out_hbm,
          tidx_v, sub_v, ubuf_v, ibuf_v, out_v, sem_u, sem_i)


def kernel(Tu, Ti, uY, iY):
    u3 = uY.reshape(NSLAB, SLAB, D)
    i3 = iY.reshape(NSLAB, SLAB, D)
    return _mf_sc(Tu.astype(jnp.int32), Ti.astype(jnp.int32), u3, i3)
